# baseline (speedup 1.0000x reference)
import math

import jax
import jax.numpy as jnp
from jax.experimental import pallas as pl
from jax.experimental.pallas import tpu as pltpu


def _pe_table(pe_dim, n):
    pos = jnp.arange(n, dtype=jnp.float32)[:, None]
    div = jnp.exp(jnp.arange(0, pe_dim, 2, dtype=jnp.float32) *
                  (-math.log(10000.0) / pe_dim))
    pe = jnp.zeros((n, pe_dim), dtype=jnp.float32)
    pe = pe.at[:, 0::2].set(jnp.sin(pos * div))
    pe = pe.at[:, 1::2].set(jnp.cos(pos * div))
    return pe


def _norm_adj(edge_index, edge_weight, num_graphs, n):
    g = edge_index[0] // n
    src = edge_index[0] % n
    dst = edge_index[1] % n
    a = jnp.zeros((num_graphs, n, n), dtype=jnp.float32)
    a = a.at[g, src, dst].add(edge_weight.astype(jnp.float32))
    a = a + jnp.eye(n, dtype=jnp.float32)[None]
    deg = a.sum(axis=2)
    d_inv = 1.0 / jnp.sqrt(jnp.maximum(deg, 1e-12))
    return a * d_inv[:, :, None] * d_inv[:, None, :]


def _fused_kernel(
    cols_ref,
    ahat_ref,
    wc1_ref,
    bc1_ref,
    wc2k_ref,
    bc2_ref,
    wg1_ref,
    bg1_ref,
    wg2_ref,
    bg2_ref,
    w1ag_ref,
    w1ac_ref,
    w1bg_ref,
    w1bc_ref,
    b1_ref,
    w2_ref,
    b2_ref,
    w3_ref,
    b3_ref,
    o_ref,
    c1pad_ref,
):
    n = ahat_ref.shape[1]
    taps = 15
    pad = (taps - 1) // 2

    c1pad_ref[...] = jnp.zeros_like(c1pad_ref)

    def one_graph(g, carry):
        cols = cols_ref[g]
        ahat = ahat_ref[g]

        c1 = jnp.maximum(
            jnp.dot(cols, wc1_ref[...], preferred_element_type=jnp.float32)
            + bc1_ref[...], 0.0)

        c1pad_ref[pad:pad + n, :] = c1
        cols2 = jnp.concatenate(
            [c1pad_ref[k:k + n, :] for k in range(taps)], axis=1)
        c2 = jnp.maximum(
            jnp.dot(cols2, wc2k_ref[...], preferred_element_type=jnp.float32)
            + bc2_ref[...], 0.0)

        hw1 = jnp.dot(cols, wg1_ref[...], preferred_element_type=jnp.float32)
        g1 = jnp.maximum(
            jnp.dot(ahat, hw1, preferred_element_type=jnp.float32)
            + bg1_ref[...], 0.0)
        hw2 = jnp.dot(g1, wg2_ref[...], preferred_element_type=jnp.float32)
        g2 = jnp.maximum(
            jnp.dot(ahat, hw2, preferred_element_type=jnp.float32)
            + bg2_ref[...], 0.0)

        am = (jnp.dot(g2, w1ag_ref[...], preferred_element_type=jnp.float32)
              + jnp.dot(c2, w1ac_ref[...], preferred_element_type=jnp.float32))
        bm = (jnp.dot(g2, w1bg_ref[...], preferred_element_type=jnp.float32)
              + jnp.dot(c2, w1bc_ref[...], preferred_element_type=jnp.float32)
              + b1_ref[...])
        am_bf = am.astype(jnp.bfloat16)
        bm_bf = bm.astype(jnp.bfloat16)

        h1 = jnp.maximum(am_bf[None, :, :] + bm_bf[:, None, :],
                         jnp.bfloat16(0.0))
        h1f = h1.reshape(n * n, h1.shape[-1])
        h2 = (jnp.dot(h1f, w2_ref[...], preferred_element_type=jnp.float32)
              + b2_ref[...])
        h2r = jnp.maximum(h2, 0.0) * w3_ref[...]
        s = (jnp.sum(h2r.reshape(n, n, h2r.shape[-1]), axis=-1)
             + b3_ref[0, 0])
        o_ref[g] = jnp.maximum(s, 0.0)
        return carry

    jax.lax.fori_loop(0, cols_ref.shape[0], one_graph, 0)


def kernel(wc1, bc1, wc2, bc2, wg1, bg1, wg2, bg2,
           w1a_g, w1a_c, w1b_g, w1b_c, b1, w2, b2, w3, b3,
           features, edge_index, edge_attr):
    res = 4
    pe_dim = 8
    B, F, L = features.shape
    N = L // res
    K = wc2.shape[0]
    KCP = wc1.shape[0]
    CD = wc2.shape[2]
    pad = (K - 1) // 2
    cin = F + pe_dim

    x = features[:, :, :N * res].reshape(B, F, N, res).mean(axis=-1)
    x = jnp.transpose(x, (0, 2, 1)).astype(jnp.float32)
    pe = _pe_table(pe_dim, N)
    x = jnp.concatenate([x, jnp.broadcast_to(pe[None], (B, N, pe_dim))],
                        axis=-1)

    xpad = jnp.pad(x, ((0, 0), (pad, pad), (0, 0)))
    cols = jnp.concatenate([xpad[:, k:k + N, :] for k in range(K)], axis=-1)
    cols = jnp.pad(cols, ((0, 0), (0, 0), (0, KCP - K * cin)))

    if edge_attr is None:
        edge_attr = jnp.ones((edge_index.shape[1],), dtype=jnp.float32)
    a_hat = _norm_adj(edge_index, edge_attr, B, N)

    wc2k = wc2.reshape(K * CD, CD)
    w1a = jnp.concatenate([w1a_g, w1a_c], axis=0)
    w1b = jnp.concatenate([w1b_g, w1b_c], axis=0)
    w2_bf = w2.astype(jnp.bfloat16)

    def full2d(arr):
        return pl.BlockSpec(arr.shape, lambda b: (0, 0))

    G = 8

    return pl.pallas_call(
        _fused_kernel,
        out_shape=jax.ShapeDtypeStruct((B, N, N), jnp.float32),
        grid=(B // G,),
        in_specs=[
            pl.BlockSpec((G, N, KCP), lambda b: (b, 0, 0)),
            pl.BlockSpec((G, N, N), lambda b: (b, 0, 0)),
            full2d(wc1), full2d(bc1),
            full2d(wc2k), full2d(bc2),
            full2d(wg1), full2d(bg1),
            full2d(wg2), full2d(bg2),
            full2d(w1a_g), full2d(w1a_c), full2d(w1b_g), full2d(w1b_c),
            full2d(b1),
            full2d(w2_bf), full2d(b2), full2d(w3),
            pl.BlockSpec(memory_space=pltpu.MemorySpace.SMEM),
        ],
        out_specs=pl.BlockSpec((G, N, N), lambda b: (b, 0, 0)),
        scratch_shapes=[pltpu.VMEM((N + 2 * pad + 2, CD), jnp.float32)],
        compiler_params=pltpu.CompilerParams(
            dimension_semantics=("parallel",),
            vmem_limit_bytes=48 * 1024 * 1024),
    )(cols, a_hat, wc1, bc1, wc2k, bc2, wg1, bg1, wg2, bg2,
      w1a_g, w1a_c, w1b_g, w1b_c, b1, w2_bf, b2, w3, b3)


# speedup vs baseline: 1.0167x; 1.0167x over previous
import math

import jax
import jax.numpy as jnp
from jax.experimental import pallas as pl
from jax.experimental.pallas import tpu as pltpu


def _pe_table(pe_dim, n):
    pos = jnp.arange(n, dtype=jnp.float32)[:, None]
    div = jnp.exp(jnp.arange(0, pe_dim, 2, dtype=jnp.float32) *
                  (-math.log(10000.0) / pe_dim))
    pe = jnp.zeros((n, pe_dim), dtype=jnp.float32)
    pe = pe.at[:, 0::2].set(jnp.sin(pos * div))
    pe = pe.at[:, 1::2].set(jnp.cos(pos * div))
    return pe


def _norm_adj(edge_index, edge_weight, num_graphs, n):
    g = edge_index[0] // n
    src = edge_index[0] % n
    dst = edge_index[1] % n
    a = jnp.zeros((num_graphs, n, n), dtype=jnp.float32)
    a = a.at[g, src, dst].add(edge_weight.astype(jnp.float32))
    a = a + jnp.eye(n, dtype=jnp.float32)[None]
    deg = a.sum(axis=2)
    d_inv = 1.0 / jnp.sqrt(jnp.maximum(deg, 1e-12))
    return a * d_inv[:, :, None] * d_inv[:, None, :]


def _fused_kernel(
    cols_ref,
    ahat_ref,
    wc1_ref,
    bc1_ref,
    wc2k_ref,
    bc2_ref,
    wg1_ref,
    bg1_ref,
    wg2_ref,
    bg2_ref,
    w1ag_ref,
    w1ac_ref,
    w1bg_ref,
    w1bc_ref,
    b1_ref,
    w2_ref,
    b2_ref,
    w3_ref,
    b3_ref,
    o_ref,
    c1pad_ref,
):
    n = ahat_ref.shape[1]
    taps = 15
    pad = (taps - 1) // 2

    c1pad_ref[...] = jnp.zeros_like(c1pad_ref)

    def one_graph(g, carry):
        cols = cols_ref[g]
        ahat = ahat_ref[g]

        c1 = jnp.maximum(
            jnp.dot(cols, wc1_ref[...], preferred_element_type=jnp.float32)
            + bc1_ref[...], 0.0)

        c1pad_ref[pad:pad + n, :] = c1
        cols2 = jnp.concatenate(
            [c1pad_ref[k:k + n, :] for k in range(taps)], axis=1)
        c2 = jnp.maximum(
            jnp.dot(cols2, wc2k_ref[...], preferred_element_type=jnp.float32)
            + bc2_ref[...], 0.0)

        hw1 = jnp.dot(cols, wg1_ref[...], preferred_element_type=jnp.float32)
        g1 = jnp.maximum(
            jnp.dot(ahat, hw1, preferred_element_type=jnp.float32)
            + bg1_ref[...], 0.0)
        hw2 = jnp.dot(g1, wg2_ref[...], preferred_element_type=jnp.float32)
        g2 = jnp.maximum(
            jnp.dot(ahat, hw2, preferred_element_type=jnp.float32)
            + bg2_ref[...], 0.0)

        am = (jnp.dot(g2, w1ag_ref[...], preferred_element_type=jnp.float32)
              + jnp.dot(c2, w1ac_ref[...], preferred_element_type=jnp.float32))
        bm = (jnp.dot(g2, w1bg_ref[...], preferred_element_type=jnp.float32)
              + jnp.dot(c2, w1bc_ref[...], preferred_element_type=jnp.float32)
              + b1_ref[...])
        am_bf = am.astype(jnp.bfloat16)
        bm_bf = bm.astype(jnp.bfloat16)

        h1 = jnp.maximum(am_bf[None, :, :] + bm_bf[:, None, :],
                         jnp.bfloat16(0.0))
        h1f = h1.reshape(n * n, h1.shape[-1])
        h2 = (jnp.dot(h1f, w2_ref[...], preferred_element_type=jnp.float32)
              + b2_ref[...])
        h2r = jnp.maximum(h2, 0.0) * w3_ref[...]
        s = (jnp.sum(h2r.reshape(n, n, h2r.shape[-1]), axis=-1)
             + b3_ref[0, 0])
        o_ref[g] = jnp.maximum(s, 0.0)
        return carry

    jax.lax.fori_loop(0, cols_ref.shape[0], one_graph, 0)


def kernel(wc1, bc1, wc2, bc2, wg1, bg1, wg2, bg2,
           w1a_g, w1a_c, w1b_g, w1b_c, b1, w2, b2, w3, b3,
           features, edge_index, edge_attr):
    res = 4
    pe_dim = 8
    B, F, L = features.shape
    N = L // res
    K = wc2.shape[0]
    KCP = wc1.shape[0]
    CD = wc2.shape[2]
    pad = (K - 1) // 2
    cin = F + pe_dim

    x = features[:, :, :N * res].reshape(B, F, N, res).mean(axis=-1)
    x = jnp.transpose(x, (0, 2, 1)).astype(jnp.float32)
    pe = _pe_table(pe_dim, N)
    x = jnp.concatenate([x, jnp.broadcast_to(pe[None], (B, N, pe_dim))],
                        axis=-1)

    xpad = jnp.pad(x, ((0, 0), (pad, pad), (0, 0)))
    cols = jnp.concatenate([xpad[:, k:k + N, :] for k in range(K)], axis=-1)
    cols = jnp.pad(cols, ((0, 0), (0, 0), (0, KCP - K * cin)))

    if edge_attr is None:
        edge_attr = jnp.ones((edge_index.shape[1],), dtype=jnp.float32)
    a_hat = _norm_adj(edge_index, edge_attr, B, N)

    wc2k = wc2.reshape(K * CD, CD)
    H2 = w2.shape[1]
    w2_bf = jnp.pad(w2, ((0, 0), (0, 128 - H2))).astype(jnp.bfloat16)
    b2 = jnp.pad(b2, ((0, 0), (0, 128 - H2)))
    w3 = jnp.pad(w3, ((0, 0), (0, 128 - H2)))

    def full2d(arr):
        return pl.BlockSpec(arr.shape, lambda b: (0, 0))

    G = 8

    return pl.pallas_call(
        _fused_kernel,
        out_shape=jax.ShapeDtypeStruct((B, N, N), jnp.float32),
        grid=(B // G,),
        in_specs=[
            pl.BlockSpec((G, N, KCP), lambda b: (b, 0, 0)),
            pl.BlockSpec((G, N, N), lambda b: (b, 0, 0)),
            full2d(wc1), full2d(bc1),
            full2d(wc2k), full2d(bc2),
            full2d(wg1), full2d(bg1),
            full2d(wg2), full2d(bg2),
            full2d(w1a_g), full2d(w1a_c), full2d(w1b_g), full2d(w1b_c),
            full2d(b1),
            full2d(w2_bf), full2d(b2), full2d(w3),
            pl.BlockSpec(memory_space=pltpu.MemorySpace.SMEM),
        ],
        out_specs=pl.BlockSpec((G, N, N), lambda b: (b, 0, 0)),
        scratch_shapes=[pltpu.VMEM((N + 2 * pad + 2, CD), jnp.float32)],
        compiler_params=pltpu.CompilerParams(
            dimension_semantics=("parallel",),
            vmem_limit_bytes=48 * 1024 * 1024),
    )(cols, a_hat, wc1, bc1, wc2k, bc2, wg1, bg1, wg2, bg2,
      w1a_g, w1a_c, w1b_g, w1b_c, b1, w2_bf, b2, w3, b3)
